# baseline (speedup 1.0000x reference)
import functools

import jax
import jax.numpy as jnp
from jax import lax
from jax.experimental import pallas as pl
from jax.experimental.pallas import tpu as pltpu
from jax.experimental.pallas import tpu_sc as plsc

_B, _S, _D = 1024, 200, 64
_V = 1000000
_NT = 20
_GOFF = 16
_GLEN = _S - _GOFF
_TAIL = _S - _NT
_PAD = 208
_C0 = 96
_C1 = _GLEN - _C0
_L = 16


@functools.cache
def _build(nc: int, ns: int):
    nw = nc * ns
    bpw = _B // nw
    mesh = plsc.VectorSubcoreMesh(
        core_axis_name="c", subcore_axis_name="s",
        num_cores=nc, num_subcores=ns)

    @functools.partial(
        pl.kernel,
        out_type=jax.ShapeDtypeStruct((_B * _S, _D), jnp.float32),
        mesh=mesh,
        scratch_types=[
            pltpu.VMEM((_PAD,), jnp.int32),
            pltpu.VMEM((_PAD,), jnp.int32),
            pltpu.VMEM((_PAD,), jnp.int32),
            pltpu.VMEM((_GLEN, 2 * _D), jnp.float32),
            pltpu.VMEM((_GLEN, _D), jnp.float32),
            pltpu.VMEM((_NT, _D), jnp.float32),
            pltpu.SemaphoreType.DMA,
        ],
        compiler_params=pltpu.CompilerParams(use_tc_tiling_on_sc=False),
    )
    def soft_embed(tok_hbm, wtp_hbm, learned_hbm, out_hbm,
                   tok_v, pidx_v, par_v, prow_v, rows_v, learned_v, sem):
        wid = lax.axis_index("s") * nc + lax.axis_index("c")
        base = wid * bpw
        pltpu.sync_copy(learned_hbm, learned_v)

        def body(i, carry):
            b = base + i
            pltpu.sync_copy(tok_hbm.at[pl.ds(b * _S + _GOFF, _GLEN)],
                            tok_v.at[pl.ds(0, _GLEN)])
            for c in range(_PAD // _L):
                t = tok_v[pl.ds(c * _L, _L)]
                pidx_v[pl.ds(c * _L, _L)] = jax.lax.shift_right_logical(t, 1)
                par_v[pl.ds(c * _L, _L)] = jax.lax.shift_left(
                    jax.lax.bitwise_and(t, 1), 6)
            cp0 = pltpu.async_copy(
                wtp_hbm.at[pidx_v.at[pl.ds(0, _C0)]],
                prow_v.at[pl.ds(0, _C0)], sem)
            cp1 = pltpu.async_copy(
                wtp_hbm.at[pidx_v.at[pl.ds(_C0, _C1)]],
                prow_v.at[pl.ds(_C0, _C1)], sem)
            pltpu.sync_copy(
                learned_v, out_hbm.at[pl.ds(b * _S, _NT)])
            cp0.wait()
            cp1.wait()

            def sel(j, carry2):
                off = par_v[pl.ds(j, _L)][0]
                for c in range(_D // _L):
                    rows_v[j, pl.ds(c * _L, _L)] = (
                        prow_v[j, pl.ds(off + c * _L, _L)])
                return carry2

            lax.fori_loop(_NT - _GOFF, _GLEN, sel, 0)
            pltpu.sync_copy(
                rows_v.at[pl.ds(_NT - _GOFF, _TAIL)],
                out_hbm.at[pl.ds(b * _S + _NT, _TAIL)])
            return carry

        lax.fori_loop(0, bpw, body, 0)

    return soft_embed


def kernel(tokens, wte_weight, learned_embedding):
    info = plsc.get_sparse_core_info()
    k = _build(info.num_cores, info.num_subcores)
    out = k(tokens.astype(jnp.int32).reshape(_B * _S),
            wte_weight.reshape(_V // 2, 2 * _D),
            learned_embedding)
    return out.reshape(_B, _S, _D)
